# Initial kernel scaffold; baseline (speedup 1.0000x reference)
import jax
import jax.numpy as jnp
from jax.experimental import pallas as pl


def kernel(x, table):
    raise NotImplementedError("write your pallas kernel here")


# speedup vs baseline: 1.0937x; 1.0937x over previous
import functools

import jax
import jax.numpy as jnp
from jax import lax
from jax.experimental import pallas as pl
from jax.experimental.pallas import tpu as pltpu
from jax.experimental.pallas import tpu_sc as plsc

BATCH = 16384
HIST = 50
EMBED_DIM = 32
TOTAL = BATCH * HIST

NUM_CORES = 2
NUM_SUBCORES = 16
NUM_WORKERS = NUM_CORES * NUM_SUBCORES
B_PER_W = TOTAL // NUM_WORKERS
CHUNK = 1024
N_CHUNKS = B_PER_W // CHUNK


def _gather_body(idx_hbm, table_hbm, out_hbm, idx_v, rows_v, sem):
    wid = lax.axis_index("s") * NUM_CORES + lax.axis_index("c")
    base = wid * B_PER_W

    def body(i, carry):
        start = base + i * CHUNK
        pltpu.sync_copy(idx_hbm.at[pl.ds(start, CHUNK)], idx_v)
        pltpu.async_copy(table_hbm.at[idx_v], rows_v, sem).wait()
        pltpu.sync_copy(rows_v, out_hbm.at[pl.ds(start, CHUNK)])
        return carry

    lax.fori_loop(0, N_CHUNKS, body, 0)


_mesh = plsc.VectorSubcoreMesh(core_axis_name="c", subcore_axis_name="s")

_gather = functools.partial(
    pl.kernel,
    mesh=_mesh,
    compiler_params=pltpu.CompilerParams(use_tc_tiling_on_sc=False),
    out_type=jax.ShapeDtypeStruct((TOTAL, EMBED_DIM), jnp.float32),
    scratch_types=[
        pltpu.VMEM((CHUNK,), jnp.int32),
        pltpu.VMEM((CHUNK, EMBED_DIM), jnp.float32),
        pltpu.SemaphoreType.DMA,
    ],
)(_gather_body)


def kernel(x, table):
    idx = x.reshape(TOTAL).astype(jnp.int32)
    out = _gather(idx, table)
    return out.reshape(BATCH, HIST, EMBED_DIM)


# speedup vs baseline: 1.1081x; 1.0131x over previous
import functools

import jax
import jax.numpy as jnp
from jax import lax
from jax.experimental import pallas as pl
from jax.experimental.pallas import tpu as pltpu
from jax.experimental.pallas import tpu_sc as plsc

BATCH = 16384
HIST = 50
EMBED_DIM = 32
TOTAL = BATCH * HIST

NUM_CORES = 2
NUM_SUBCORES = 16
NUM_WORKERS = NUM_CORES * NUM_SUBCORES
B_PER_W = TOTAL // NUM_WORKERS
CHUNK = 1280
N_CHUNKS = B_PER_W // CHUNK


def _gather_body(idx_hbm, table_hbm, out_hbm, idx_v, rows0, rows1,
                 gs0, gs1, ws0, ws1):
    wid = lax.axis_index("s") * NUM_CORES + lax.axis_index("c")
    base = wid * B_PER_W
    pltpu.sync_copy(idx_hbm.at[pl.ds(base, B_PER_W)], idx_v)

    rows = (rows0, rows1)
    gs = (gs0, gs1)
    ws = (ws0, ws1)

    def gather_desc(i, b):
        return pltpu.make_async_copy(
            table_hbm.at[idx_v.at[pl.ds(i * CHUNK, CHUNK)]], rows[b], gs[b])

    def write_desc(i, b):
        return pltpu.make_async_copy(
            rows[b], out_hbm.at[pl.ds(base + i * CHUNK, CHUNK)], ws[b])

    gather_desc(0, 0).start()
    gather_desc(1, 1).start()

    def body(g, carry):
        i0 = g * 2
        for b in range(2):
            gather_desc(i0 + b, b).wait()
            write_desc(i0 + b, b).start()

        @pl.when(i0 + 2 < N_CHUNKS)
        def _prefetch():
            for b in range(2):
                write_desc(i0 + b, b).wait()
                gather_desc(i0 + 2 + b, b).start()

        return carry

    lax.fori_loop(0, N_CHUNKS // 2, body, 0)
    write_desc(N_CHUNKS - 2, 0).wait()
    write_desc(N_CHUNKS - 1, 1).wait()


_mesh = plsc.VectorSubcoreMesh(core_axis_name="c", subcore_axis_name="s")

_gather = functools.partial(
    pl.kernel,
    mesh=_mesh,
    compiler_params=pltpu.CompilerParams(use_tc_tiling_on_sc=False),
    out_type=jax.ShapeDtypeStruct((TOTAL, EMBED_DIM), jnp.float32),
    scratch_types=[
        pltpu.VMEM((B_PER_W,), jnp.int32),
        pltpu.VMEM((CHUNK, EMBED_DIM), jnp.float32),
        pltpu.VMEM((CHUNK, EMBED_DIM), jnp.float32),
        pltpu.SemaphoreType.DMA,
        pltpu.SemaphoreType.DMA,
        pltpu.SemaphoreType.DMA,
        pltpu.SemaphoreType.DMA,
    ],
)(_gather_body)


def kernel(x, table):
    idx = x.reshape(TOTAL).astype(jnp.int32)
    out = _gather(idx, table)
    return out.reshape(BATCH, HIST, EMBED_DIM)


# speedup vs baseline: 1.1086x; 1.0005x over previous
import functools

import jax
import jax.numpy as jnp
from jax import lax
from jax.experimental import pallas as pl
from jax.experimental.pallas import tpu as pltpu
from jax.experimental.pallas import tpu_sc as plsc

BATCH = 16384
HIST = 50
EMBED_DIM = 32
TOTAL = BATCH * HIST

NUM_CORES = 2
NUM_SUBCORES = 16
NUM_WORKERS = NUM_CORES * NUM_SUBCORES
B_PER_W = TOTAL // NUM_WORKERS
CHUNK = 1280
N_CHUNKS = B_PER_W // CHUNK
SUBSTREAMS = 4
SUBCHUNK = CHUNK // SUBSTREAMS


def _gather_body(idx_hbm, table_hbm, out_hbm, idx_v, rows0, rows1,
                 gs0, gs1, ws0, ws1):
    wid = lax.axis_index("s") * NUM_CORES + lax.axis_index("c")
    base = wid * B_PER_W
    pltpu.sync_copy(idx_hbm.at[pl.ds(base, B_PER_W)], idx_v)

    rows = (rows0, rows1)
    gs = (gs0, gs1)
    ws = (ws0, ws1)

    def gather_descs(i, b):
        return [
            pltpu.make_async_copy(
                table_hbm.at[idx_v.at[pl.ds(i * CHUNK + s * SUBCHUNK,
                                            SUBCHUNK)]],
                rows[b].at[pl.ds(s * SUBCHUNK, SUBCHUNK)],
                gs[b])
            for s in range(SUBSTREAMS)
        ]

    def start_gather(i, b):
        for d in gather_descs(i, b):
            d.start()

    def wait_gather(i, b):
        for d in gather_descs(i, b):
            d.wait()

    def write_desc(i, b):
        return pltpu.make_async_copy(
            rows[b], out_hbm.at[pl.ds(base + i * CHUNK, CHUNK)], ws[b])

    start_gather(0, 0)
    start_gather(1, 1)

    def body(g, carry):
        i0 = g * 2
        for b in range(2):
            wait_gather(i0 + b, b)
            write_desc(i0 + b, b).start()

        @pl.when(i0 + 2 < N_CHUNKS)
        def _prefetch():
            for b in range(2):
                write_desc(i0 + b, b).wait()
                start_gather(i0 + 2 + b, b)

        return carry

    lax.fori_loop(0, N_CHUNKS // 2, body, 0)
    write_desc(N_CHUNKS - 2, 0).wait()
    write_desc(N_CHUNKS - 1, 1).wait()


_mesh = plsc.VectorSubcoreMesh(core_axis_name="c", subcore_axis_name="s")

_gather = functools.partial(
    pl.kernel,
    mesh=_mesh,
    compiler_params=pltpu.CompilerParams(use_tc_tiling_on_sc=False),
    out_type=jax.ShapeDtypeStruct((TOTAL, EMBED_DIM), jnp.float32),
    scratch_types=[
        pltpu.VMEM((B_PER_W,), jnp.int32),
        pltpu.VMEM((CHUNK, EMBED_DIM), jnp.float32),
        pltpu.VMEM((CHUNK, EMBED_DIM), jnp.float32),
        pltpu.SemaphoreType.DMA,
        pltpu.SemaphoreType.DMA,
        pltpu.SemaphoreType.DMA,
        pltpu.SemaphoreType.DMA,
    ],
)(_gather_body)


def kernel(x, table):
    idx = x.reshape(TOTAL).astype(jnp.int32)
    out = _gather(idx, table)
    return out.reshape(BATCH, HIST, EMBED_DIM)


# speedup vs baseline: 1.7977x; 1.6215x over previous
import functools

import jax
import jax.numpy as jnp
from jax import lax
from jax.experimental import pallas as pl
from jax.experimental.pallas import tpu as pltpu
from jax.experimental.pallas import tpu_sc as plsc

BATCH = 16384
HIST = 50
EMBED_DIM = 32
TOTAL = BATCH * HIST

NUM_CORES = 2
NUM_SUBCORES = 16
NUM_WORKERS = NUM_CORES * NUM_SUBCORES
B_PER_W = TOTAL // NUM_WORKERS
ROWS_PER_W = BATCH // NUM_WORKERS
CHUNK_ROWS = 16
CHUNK = CHUNK_ROWS * HIST
N_CHUNKS = ROWS_PER_W // CHUNK_ROWS


def _gather_body(idx_hbm, table_hbm, out_hbm, idx_v, rows0, rows1,
                 gs0, gs1, ws0, ws1):
    wid = lax.axis_index("s") * NUM_CORES + lax.axis_index("c")
    base = wid * B_PER_W
    row_base = wid * ROWS_PER_W
    pltpu.sync_copy(idx_hbm.at[pl.ds(base, B_PER_W)], idx_v)

    rows = (rows0, rows1)
    gs = (gs0, gs1)
    ws = (ws0, ws1)

    def gather_desc(i, b):
        return pltpu.make_async_copy(
            table_hbm.at[idx_v.at[pl.ds(i * CHUNK, CHUNK)]], rows[b], gs[b])

    def row_write_desc(i, b, r):
        return pltpu.make_async_copy(
            rows[b].at[pl.ds(r * HIST, HIST)],
            out_hbm.at[row_base + i * CHUNK_ROWS + r],
            ws[b])

    def start_writes(i, b):
        for r in range(CHUNK_ROWS):
            row_write_desc(i, b, r).start()

    def wait_writes(i, b):
        for r in range(CHUNK_ROWS):
            row_write_desc(i, b, r).wait()

    gather_desc(0, 0).start()
    gather_desc(1, 1).start()

    def body(g, carry):
        i0 = g * 2
        for b in range(2):
            gather_desc(i0 + b, b).wait()
            start_writes(i0 + b, b)

        @pl.when(i0 + 2 < N_CHUNKS)
        def _prefetch():
            for b in range(2):
                wait_writes(i0 + b, b)
                gather_desc(i0 + 2 + b, b).start()

        return carry

    lax.fori_loop(0, N_CHUNKS // 2, body, 0)
    wait_writes(N_CHUNKS - 2, 0)
    wait_writes(N_CHUNKS - 1, 1)


_mesh = plsc.VectorSubcoreMesh(core_axis_name="c", subcore_axis_name="s")

_gather = functools.partial(
    pl.kernel,
    mesh=_mesh,
    compiler_params=pltpu.CompilerParams(use_tc_tiling_on_sc=False),
    out_type=jax.ShapeDtypeStruct((BATCH, HIST, EMBED_DIM), jnp.float32),
    scratch_types=[
        pltpu.VMEM((B_PER_W,), jnp.int32),
        pltpu.VMEM((CHUNK, EMBED_DIM), jnp.float32),
        pltpu.VMEM((CHUNK, EMBED_DIM), jnp.float32),
        pltpu.SemaphoreType.DMA,
        pltpu.SemaphoreType.DMA,
        pltpu.SemaphoreType.DMA,
        pltpu.SemaphoreType.DMA,
    ],
)(_gather_body)


def kernel(x, table):
    return _gather(x.reshape(TOTAL).astype(jnp.int32), table)


# speedup vs baseline: 1.8023x; 1.0025x over previous
import functools

import jax
import jax.numpy as jnp
from jax import lax
from jax.experimental import pallas as pl
from jax.experimental.pallas import tpu as pltpu
from jax.experimental.pallas import tpu_sc as plsc

BATCH = 16384
HIST = 50
EMBED_DIM = 32
TOTAL = BATCH * HIST

NUM_CORES = 2
NUM_SUBCORES = 16
NUM_WORKERS = NUM_CORES * NUM_SUBCORES
B_PER_W = TOTAL // NUM_WORKERS
ROWS_PER_W = BATCH // NUM_WORKERS
CHUNK_ROWS = 32
CHUNK = CHUNK_ROWS * HIST
N_CHUNKS = ROWS_PER_W // CHUNK_ROWS


def _gather_body(idx_hbm, table_hbm, out_hbm, idx_v, rows0, rows1,
                 gs0, gs1, ws0, ws1):
    wid = lax.axis_index("s") * NUM_CORES + lax.axis_index("c")
    base = wid * B_PER_W
    row_base = wid * ROWS_PER_W
    pltpu.sync_copy(idx_hbm.at[pl.ds(base, B_PER_W)], idx_v)

    rows = (rows0, rows1)
    gs = (gs0, gs1)
    ws = (ws0, ws1)

    def gather_desc(i, b):
        return pltpu.make_async_copy(
            table_hbm.at[idx_v.at[pl.ds(i * CHUNK, CHUNK)]], rows[b], gs[b])

    def row_write_desc(i, b, r):
        return pltpu.make_async_copy(
            rows[b].at[pl.ds(r * HIST, HIST)],
            out_hbm.at[row_base + i * CHUNK_ROWS + r],
            ws[b])

    def start_writes(i, b):
        for r in range(CHUNK_ROWS):
            row_write_desc(i, b, r).start()

    def wait_writes(i, b):
        for r in range(CHUNK_ROWS):
            row_write_desc(i, b, r).wait()

    gather_desc(0, 0).start()
    gather_desc(1, 1).start()

    def body(g, carry):
        i0 = g * 2
        for b in range(2):
            gather_desc(i0 + b, b).wait()
            start_writes(i0 + b, b)

        @pl.when(i0 + 2 < N_CHUNKS)
        def _prefetch():
            for b in range(2):
                wait_writes(i0 + b, b)
                gather_desc(i0 + 2 + b, b).start()

        return carry

    lax.fori_loop(0, N_CHUNKS // 2, body, 0)
    wait_writes(N_CHUNKS - 2, 0)
    wait_writes(N_CHUNKS - 1, 1)


_mesh = plsc.VectorSubcoreMesh(core_axis_name="c", subcore_axis_name="s")

_gather = functools.partial(
    pl.kernel,
    mesh=_mesh,
    compiler_params=pltpu.CompilerParams(use_tc_tiling_on_sc=False),
    out_type=jax.ShapeDtypeStruct((BATCH, HIST, EMBED_DIM), jnp.float32),
    scratch_types=[
        pltpu.VMEM((B_PER_W,), jnp.int32),
        pltpu.VMEM((CHUNK, EMBED_DIM), jnp.float32),
        pltpu.VMEM((CHUNK, EMBED_DIM), jnp.float32),
        pltpu.SemaphoreType.DMA,
        pltpu.SemaphoreType.DMA,
        pltpu.SemaphoreType.DMA,
        pltpu.SemaphoreType.DMA,
    ],
)(_gather_body)


def kernel(x, table):
    return _gather(x.reshape(TOTAL).astype(jnp.int32), table)
